# baseline (speedup 1.0000x reference)
import functools

import jax
import jax.numpy as jnp
from jax import lax
from jax.experimental import pallas as pl
from jax.experimental.pallas import tpu as pltpu
from jax.experimental.pallas import tpu_sc as plsc

B = 16384
D = 64
W = 2 * D
NE = 1000000
NR = 1000
NW = 32
BPW = B // NW
CH = 128
NCHUNK = BPW // CH
NROUND = 2
RCH = NCHUNK // NROUND
RROWS = RCH * CH
L = 16


def _transe_body(hp_hbm, rp_hbm, tp_hbm, hq_hbm, rq_hbm, tq_hbm,
                 ent_hbm, rel_hbm, out_hbm,
                 hp, rp, tp, hq, rq, tq, hrows, rrows, trows, outv, sem):
    wid = lax.axis_index("s") * 2 + lax.axis_index("c")
    blk = pl.ds(wid * NCHUNK, NCHUNK)

    pltpu.sync_copy(hp_hbm.at[blk], hp)
    pltpu.sync_copy(rp_hbm.at[blk], rp)
    pltpu.sync_copy(tp_hbm.at[blk], tp)
    pltpu.sync_copy(hq_hbm.at[blk], hq)
    pltpu.sync_copy(rq_hbm.at[blk], rq)
    pltpu.sync_copy(tq_hbm.at[blk], tq)

    lane = lax.iota(jnp.int32, L)

    for rnd in range(NROUND):
        copies = []
        for c in range(RCH):
            dst = pl.ds(c * CH, CH)
            copies.append(pltpu.async_copy(
                ent_hbm.at[hp.at[rnd * RCH + c]], hrows.at[dst], sem))
            copies.append(pltpu.async_copy(
                rel_hbm.at[rp.at[rnd * RCH + c]], rrows.at[dst], sem))
            copies.append(pltpu.async_copy(
                ent_hbm.at[tp.at[rnd * RCH + c]], trows.at[dst], sem))
        for cp in copies:
            cp.wait()

        def group_body(g, carry):
            cc = rnd * RCH + g // 8
            po = (g % 8) * L
            psl = pl.ds(po, L)
            hqv = hq[cc, psl]
            rqv = rq[cc, psl]
            tqv = tq[cc, psl]
            svec = jnp.zeros((L,), jnp.float32)
            for k in range(L):
                i = g * L + k
                offh = hqv[k] * D
                offr = rqv[k] * D
                offt = tqv[k] * D
                acc = jnp.zeros((L,), jnp.float32)
                for j in range(D // L):
                    h = hrows[i, pl.ds(offh + j * L, L)]
                    r = rrows[i, pl.ds(offr + j * L, L)]
                    t = trows[i, pl.ds(offt + j * L, L)]
                    d = (h + r) - t
                    acc = acc + d * d
                svec = jnp.where(lane == k, jnp.sum(acc), svec)
            x = svec + 1e-12
            xi = plsc.bitcast(x, jnp.int32)
            yi = jnp.full((L,), 0x5F3759DF, jnp.int32) - (xi >> 1)
            y = plsc.bitcast(yi, jnp.float32)
            for _ in range(3):
                y = y * (1.5 - 0.5 * x * y * y)
            outv[pl.ds(rnd * RROWS + g * L, L)] = -(x * y)
            return carry

        lax.fori_loop(0, RROWS // L, group_body, 0)

    pltpu.sync_copy(outv, out_hbm.at[pl.ds(wid * BPW, BPW)])


@jax.jit
def _transe_sc(hp, rp, tp, hq, rq, tq, ent2, rel2):
    mesh = plsc.VectorSubcoreMesh(core_axis_name="c", subcore_axis_name="s")
    return pl.kernel(
        _transe_body,
        mesh=mesh,
        compiler_params=pltpu.CompilerParams(
            needs_layout_passes=False, use_tc_tiling_on_sc=False),
        out_type=jax.ShapeDtypeStruct((B,), jnp.float32),
        scratch_types=[
            pltpu.VMEM((NCHUNK, CH), jnp.int32),
            pltpu.VMEM((NCHUNK, CH), jnp.int32),
            pltpu.VMEM((NCHUNK, CH), jnp.int32),
            pltpu.VMEM((NCHUNK, CH), jnp.int32),
            pltpu.VMEM((NCHUNK, CH), jnp.int32),
            pltpu.VMEM((NCHUNK, CH), jnp.int32),
            pltpu.VMEM((RROWS, W), jnp.float32),
            pltpu.VMEM((RROWS, W), jnp.float32),
            pltpu.VMEM((RROWS, W), jnp.float32),
            pltpu.VMEM((BPW,), jnp.float32),
            pltpu.SemaphoreType.DMA,
        ],
    )(hp, rp, tp, hq, rq, tq, ent2, rel2)


def kernel(heads, relations, tails, entity_embed, relation_embed):
    ent2 = entity_embed.reshape(NE // 2, W)
    rel2 = relation_embed.reshape(NR // 2, W)
    h32 = heads.astype(jnp.int32)
    r32 = relations.astype(jnp.int32)
    t32 = tails.astype(jnp.int32)
    hp = (h32 >> 1).reshape(B // CH, CH)
    rp = (r32 >> 1).reshape(B // CH, CH)
    tp = (t32 >> 1).reshape(B // CH, CH)
    hq = (h32 & 1).reshape(B // CH, CH)
    rq = (r32 & 1).reshape(B // CH, CH)
    tq = (t32 & 1).reshape(B // CH, CH)
    return _transe_sc(hp, rp, tp, hq, rq, tq, ent2, rel2)


# speedup vs baseline: 1.0019x; 1.0019x over previous
import functools

import jax
import jax.numpy as jnp
from jax import lax
from jax.experimental import pallas as pl
from jax.experimental.pallas import tpu as pltpu
from jax.experimental.pallas import tpu_sc as plsc

B = 16384
D = 64
W = 2 * D
NE = 1000000
NR = 1000
NW = 32
BPW = B // NW
CH = 128
NCHUNK = BPW // CH
NROUND = 2
RCH = NCHUNK // NROUND
RROWS = RCH * CH
NT = 3 * NCHUNK
L = 16


def _transe_body(idx_hbm, ent_hbm, rel_hbm, out_hbm,
                 idxraw, idxpair, hrows, rrows, trows, outv, sem):
    wid = lax.axis_index("s") * 2 + lax.axis_index("c")

    pltpu.sync_copy(idx_hbm.at[wid], idxraw)

    def pair_body(m, carry):
        row = m // 8
        sl = pl.ds((m % 8) * L, L)
        idxpair[row, sl] = idxraw[row, sl] >> 1
        return carry

    lax.fori_loop(0, NT * 8, pair_body, 0)

    lane = lax.iota(jnp.int32, L)

    for rnd in range(NROUND):
        copies = []
        for c in range(RCH):
            cc = rnd * RCH + c
            dst = pl.ds(c * CH, CH)
            copies.append(pltpu.async_copy(
                ent_hbm.at[idxpair.at[cc]], hrows.at[dst], sem))
            copies.append(pltpu.async_copy(
                rel_hbm.at[idxpair.at[NCHUNK + cc]], rrows.at[dst], sem))
            copies.append(pltpu.async_copy(
                ent_hbm.at[idxpair.at[2 * NCHUNK + cc]], trows.at[dst], sem))
        for cp in copies:
            cp.wait()

        def group_body(g, carry):
            cc = rnd * RCH + g // 8
            psl = pl.ds((g % 8) * L, L)
            hqv = idxraw[cc, psl] & 1
            rqv = idxraw[NCHUNK + cc, psl] & 1
            tqv = idxraw[2 * NCHUNK + cc, psl] & 1
            svec = jnp.zeros((L,), jnp.float32)
            for k in range(L):
                i = g * L + k
                offh = hqv[k] * D
                offr = rqv[k] * D
                offt = tqv[k] * D
                acc = jnp.zeros((L,), jnp.float32)
                for j in range(D // L):
                    h = hrows[i, pl.ds(offh + j * L, L)]
                    r = rrows[i, pl.ds(offr + j * L, L)]
                    t = trows[i, pl.ds(offt + j * L, L)]
                    d = (h + r) - t
                    acc = acc + d * d
                svec = jnp.where(lane == k, jnp.sum(acc), svec)
            x = svec + 1e-12
            xi = plsc.bitcast(x, jnp.int32)
            yi = jnp.full((L,), 0x5F3759DF, jnp.int32) - (xi >> 1)
            y = plsc.bitcast(yi, jnp.float32)
            for _ in range(3):
                y = y * (1.5 - 0.5 * x * y * y)
            outv[pl.ds(rnd * RROWS + g * L, L)] = -(x * y)
            return carry

        lax.fori_loop(0, RROWS // L, group_body, 0)

    pltpu.sync_copy(outv, out_hbm.at[pl.ds(wid * BPW, BPW)])


@jax.jit
def _transe_sc(idx_all, ent2, rel2):
    mesh = plsc.VectorSubcoreMesh(core_axis_name="c", subcore_axis_name="s")
    return pl.kernel(
        _transe_body,
        mesh=mesh,
        compiler_params=pltpu.CompilerParams(
            needs_layout_passes=False, use_tc_tiling_on_sc=True),
        out_type=jax.ShapeDtypeStruct((B,), jnp.float32),
        scratch_types=[
            pltpu.VMEM((NT, CH), jnp.int32),
            pltpu.VMEM((NT, CH), jnp.int32),
            pltpu.VMEM((RROWS, W), jnp.float32),
            pltpu.VMEM((RROWS, W), jnp.float32),
            pltpu.VMEM((RROWS, W), jnp.float32),
            pltpu.VMEM((BPW,), jnp.float32),
            pltpu.SemaphoreType.DMA,
        ],
    )(idx_all, ent2, rel2)


def kernel(heads, relations, tails, entity_embed, relation_embed):
    ent2 = entity_embed.reshape(NE // 2, W)
    rel2 = relation_embed.reshape(NR // 2, W)
    h32 = heads.astype(jnp.int32).reshape(NW, NCHUNK, CH)
    r32 = relations.astype(jnp.int32).reshape(NW, NCHUNK, CH)
    t32 = tails.astype(jnp.int32).reshape(NW, NCHUNK, CH)
    idx_all = jnp.concatenate([h32, r32, t32], axis=1)
    return _transe_sc(idx_all, ent2, rel2)


# speedup vs baseline: 2.1606x; 2.1566x over previous
import functools

import jax
import jax.numpy as jnp
from jax import lax
from jax.experimental import pallas as pl
from jax.experimental.pallas import tpu as pltpu
from jax.experimental.pallas import tpu_sc as plsc

B = 16384
D = 64
W = 2 * D
NE = 1000000
NR = 1000
NW = 32
BPW = B // NW
CH = 128
NCHUNK = BPW // CH
NROUND = 2
RCH = NCHUNK // NROUND
RROWS = RCH * CH
NT = 3 * NCHUNK
L = 16

TB = 4096
TGRID = (NE + 2 * TB - 1) // (2 * TB)
NP = TGRID * TB


def _fold_body(lo_ref, hi_ref, out_ref):
    lo = lo_ref[...]
    hi = hi_ref[...]
    out_ref[...] = jnp.concatenate([lo.T, hi.T], axis=1)


@jax.jit
def _fold_tc(entt):
    return pl.pallas_call(
        _fold_body,
        grid=(TGRID,),
        in_specs=[
            pl.BlockSpec((D, TB), lambda i: (0, 2 * i)),
            pl.BlockSpec(
                (D, TB),
                lambda i: (0, jnp.minimum(2 * i + 1, NE // TB))),
        ],
        out_specs=pl.BlockSpec((TB, W), lambda i: (i, 0)),
        out_shape=jax.ShapeDtypeStruct((NP, W), jnp.float32),
    )(entt, entt)


def _transe_body(idx_hbm, ent_hbm, rel_hbm, out_hbm,
                 idxraw, idxpair, hrows, rrows, trows, outv, sem):
    wid = lax.axis_index("s") * 2 + lax.axis_index("c")

    pltpu.sync_copy(idx_hbm.at[wid], idxraw)

    def pair_ent(m, carry):
        row = m // 8 + (m // (8 * NCHUNK)) * NCHUNK
        sl = pl.ds((m % 8) * L, L)
        v = idxraw[row, sl]
        idxpair[row, sl] = ((v >> 13) << 12) | (v & 4095)
        return carry

    lax.fori_loop(0, 2 * NCHUNK * 8, pair_ent, 0)

    def pair_rel(m, carry):
        row = NCHUNK + m // 8
        sl = pl.ds((m % 8) * L, L)
        idxpair[row, sl] = idxraw[row, sl] >> 1
        return carry

    lax.fori_loop(0, NCHUNK * 8, pair_rel, 0)

    lane = lax.iota(jnp.int32, L)

    for rnd in range(NROUND):
        copies = []
        for c in range(RCH):
            cc = rnd * RCH + c
            dst = pl.ds(c * CH, CH)
            copies.append(pltpu.async_copy(
                ent_hbm.at[idxpair.at[cc]], hrows.at[dst], sem))
            copies.append(pltpu.async_copy(
                rel_hbm.at[idxpair.at[NCHUNK + cc]], rrows.at[dst], sem))
            copies.append(pltpu.async_copy(
                ent_hbm.at[idxpair.at[2 * NCHUNK + cc]], trows.at[dst], sem))
        for cp in copies:
            cp.wait()

        def group_body(g, carry):
            cc = rnd * RCH + g // 8
            psl = pl.ds((g % 8) * L, L)
            hqv = (idxraw[cc, psl] >> 12) & 1
            rqv = idxraw[NCHUNK + cc, psl] & 1
            tqv = (idxraw[2 * NCHUNK + cc, psl] >> 12) & 1
            svec = jnp.zeros((L,), jnp.float32)
            for k in range(L):
                i = g * L + k
                offh = hqv[k] * D
                offr = rqv[k] * D
                offt = tqv[k] * D
                acc = jnp.zeros((L,), jnp.float32)
                for j in range(D // L):
                    h = hrows[i, pl.ds(offh + j * L, L)]
                    r = rrows[i, pl.ds(offr + j * L, L)]
                    t = trows[i, pl.ds(offt + j * L, L)]
                    d = (h + r) - t
                    acc = acc + d * d
                svec = jnp.where(lane == k, jnp.sum(acc), svec)
            x = svec + 1e-12
            xi = plsc.bitcast(x, jnp.int32)
            yi = jnp.full((L,), 0x5F3759DF, jnp.int32) - (xi >> 1)
            y = plsc.bitcast(yi, jnp.float32)
            for _ in range(3):
                y = y * (1.5 - 0.5 * x * y * y)
            outv[pl.ds(rnd * RROWS + g * L, L)] = -(x * y)
            return carry

        lax.fori_loop(0, RROWS // L, group_body, 0)

    pltpu.sync_copy(outv, out_hbm.at[pl.ds(wid * BPW, BPW)])


@jax.jit
def _transe_sc(idx_all, ent2, rel2):
    mesh = plsc.VectorSubcoreMesh(core_axis_name="c", subcore_axis_name="s")
    return pl.kernel(
        _transe_body,
        mesh=mesh,
        compiler_params=pltpu.CompilerParams(
            needs_layout_passes=False, use_tc_tiling_on_sc=True),
        out_type=jax.ShapeDtypeStruct((B,), jnp.float32),
        scratch_types=[
            pltpu.VMEM((NT, CH), jnp.int32),
            pltpu.VMEM((NT, CH), jnp.int32),
            pltpu.VMEM((RROWS, W), jnp.float32),
            pltpu.VMEM((RROWS, W), jnp.float32),
            pltpu.VMEM((RROWS, W), jnp.float32),
            pltpu.VMEM((BPW,), jnp.float32),
            pltpu.SemaphoreType.DMA,
        ],
    )(idx_all, ent2, rel2)


def kernel(heads, relations, tails, entity_embed, relation_embed):
    ent2 = _fold_tc(entity_embed.T)
    rel2 = relation_embed.reshape(NR // 2, W)
    h32 = heads.astype(jnp.int32).reshape(NW, NCHUNK, CH)
    r32 = relations.astype(jnp.int32).reshape(NW, NCHUNK, CH)
    t32 = tails.astype(jnp.int32).reshape(NW, NCHUNK, CH)
    idx_all = jnp.concatenate([h32, r32, t32], axis=1)
    return _transe_sc(idx_all, ent2, rel2)


# speedup vs baseline: 3.1016x; 1.4355x over previous
import functools

import jax
import jax.numpy as jnp
from jax import lax
from jax.experimental import pallas as pl
from jax.experimental.pallas import tpu as pltpu
from jax.experimental.pallas import tpu_sc as plsc

B = 16384
D = 64
W = 2 * D
NE = 1000000
NR = 1000
NW = 32
BPW = B // NW
CH = 128
NCHUNK = BPW // CH
NROUND = 2
RCH = NCHUNK // NROUND
RROWS = RCH * CH
NT = 3 * NCHUNK
L = 16

TB = 8192
TGRID = (NE + 2 * TB - 1) // (2 * TB)
NP = TGRID * TB


def _fold_body(lo_ref, hi_ref, out_ref):
    z = jnp.concatenate([lo_ref[...], hi_ref[...]], axis=0)
    out_ref[...] = z.T


@jax.jit
def _fold_tc(entt):
    return pl.pallas_call(
        _fold_body,
        grid=(TGRID,),
        in_specs=[
            pl.BlockSpec((D, TB), lambda i: (0, 2 * i)),
            pl.BlockSpec(
                (D, TB),
                lambda i: (0, jnp.minimum(2 * i + 1, NE // TB))),
        ],
        out_specs=pl.BlockSpec((TB, W), lambda i: (i, 0)),
        out_shape=jax.ShapeDtypeStruct((NP, W), jnp.float32),
    )(entt, entt)


def _transe_body(idx_hbm, ent_hbm, rel_hbm, out_hbm,
                 idxraw, idxpair, hrows, rrows, trows, outv, sem):
    wid = lax.axis_index("s") * 2 + lax.axis_index("c")

    pltpu.sync_copy(idx_hbm.at[wid], idxraw)

    def pair_ent(m, carry):
        row = m // 8 + (m // (8 * NCHUNK)) * NCHUNK
        sl = pl.ds((m % 8) * L, L)
        v = idxraw[row, sl]
        idxpair[row, sl] = ((v >> 14) << 13) | (v & 8191)
        return carry

    lax.fori_loop(0, 2 * NCHUNK * 8, pair_ent, 0)

    def pair_rel(m, carry):
        row = NCHUNK + m // 8
        sl = pl.ds((m % 8) * L, L)
        idxpair[row, sl] = idxraw[row, sl] >> 1
        return carry

    lax.fori_loop(0, NCHUNK * 8, pair_rel, 0)

    lane = lax.iota(jnp.int32, L)

    for rnd in range(NROUND):
        copies = []
        for c in range(RCH):
            cc = rnd * RCH + c
            dst = pl.ds(c * CH, CH)
            copies.append(pltpu.async_copy(
                ent_hbm.at[idxpair.at[cc]], hrows.at[dst], sem))
            copies.append(pltpu.async_copy(
                rel_hbm.at[idxpair.at[NCHUNK + cc]], rrows.at[dst], sem))
            copies.append(pltpu.async_copy(
                ent_hbm.at[idxpair.at[2 * NCHUNK + cc]], trows.at[dst], sem))
        for cp in copies:
            cp.wait()

        def group_body(g, carry):
            cc = rnd * RCH + g // 8
            psl = pl.ds((g % 8) * L, L)
            hqv = (idxraw[cc, psl] >> 13) & 1
            rqv = idxraw[NCHUNK + cc, psl] & 1
            tqv = (idxraw[2 * NCHUNK + cc, psl] >> 13) & 1
            svec = jnp.zeros((L,), jnp.float32)
            for k in range(L):
                i = g * L + k
                offh = hqv[k] * D
                offr = rqv[k] * D
                offt = tqv[k] * D
                acc = jnp.zeros((L,), jnp.float32)
                for j in range(D // L):
                    h = hrows[i, pl.ds(offh + j * L, L)]
                    r = rrows[i, pl.ds(offr + j * L, L)]
                    t = trows[i, pl.ds(offt + j * L, L)]
                    d = (h + r) - t
                    acc = acc + d * d
                svec = jnp.where(lane == k, jnp.sum(acc), svec)
            x = svec + 1e-12
            xi = plsc.bitcast(x, jnp.int32)
            yi = jnp.full((L,), 0x5F3759DF, jnp.int32) - (xi >> 1)
            y = plsc.bitcast(yi, jnp.float32)
            for _ in range(3):
                y = y * (1.5 - 0.5 * x * y * y)
            outv[pl.ds(rnd * RROWS + g * L, L)] = -(x * y)
            return carry

        lax.fori_loop(0, RROWS // L, group_body, 0)

    pltpu.sync_copy(outv, out_hbm.at[pl.ds(wid * BPW, BPW)])


@jax.jit
def _transe_sc(idx_all, ent2, rel2):
    mesh = plsc.VectorSubcoreMesh(core_axis_name="c", subcore_axis_name="s")
    return pl.kernel(
        _transe_body,
        mesh=mesh,
        compiler_params=pltpu.CompilerParams(
            needs_layout_passes=False, use_tc_tiling_on_sc=True),
        out_type=jax.ShapeDtypeStruct((B,), jnp.float32),
        scratch_types=[
            pltpu.VMEM((NT, CH), jnp.int32),
            pltpu.VMEM((NT, CH), jnp.int32),
            pltpu.VMEM((RROWS, W), jnp.float32),
            pltpu.VMEM((RROWS, W), jnp.float32),
            pltpu.VMEM((RROWS, W), jnp.float32),
            pltpu.VMEM((BPW,), jnp.float32),
            pltpu.SemaphoreType.DMA,
        ],
    )(idx_all, ent2, rel2)


def kernel(heads, relations, tails, entity_embed, relation_embed):
    ent2 = _fold_tc(entity_embed.T)
    rel2 = relation_embed.reshape(NR // 2, W)
    h32 = heads.astype(jnp.int32).reshape(NW, NCHUNK, CH)
    r32 = relations.astype(jnp.int32).reshape(NW, NCHUNK, CH)
    t32 = tails.astype(jnp.int32).reshape(NW, NCHUNK, CH)
    idx_all = jnp.concatenate([h32, r32, t32], axis=1)
    return _transe_sc(idx_all, ent2, rel2)


# speedup vs baseline: 3.1903x; 1.0286x over previous
import functools

import jax
import jax.numpy as jnp
from jax import lax
from jax.experimental import pallas as pl
from jax.experimental.pallas import tpu as pltpu
from jax.experimental.pallas import tpu_sc as plsc

B = 16384
D = 64
W = 2 * D
NE = 1000000
NR = 1000
NW = 32
BPW = B // NW
CH = 128
NCHUNK = BPW // CH
NROUND = 2
RCH = NCHUNK // NROUND
RROWS = RCH * CH
NT = 3 * NCHUNK
L = 16

TB = 16384
TBL = TB.bit_length() - 1
TGRID = (NE + 2 * TB - 1) // (2 * TB)
NP = TGRID * TB


def _fold_body(lo_ref, hi_ref, out_ref):
    z = jnp.concatenate([lo_ref[...], hi_ref[...]], axis=0)
    out_ref[...] = z.T


@jax.jit
def _fold_tc(entt):
    return pl.pallas_call(
        _fold_body,
        grid=(TGRID,),
        in_specs=[
            pl.BlockSpec((D, TB), lambda i: (0, 2 * i)),
            pl.BlockSpec(
                (D, TB),
                lambda i: (0, jnp.minimum(2 * i + 1, NE // TB))),
        ],
        out_specs=pl.BlockSpec((TB, W), lambda i: (i, 0)),
        out_shape=jax.ShapeDtypeStruct((NP, W), jnp.float32),
    )(entt, entt)


def _transe_body(idx_hbm, ent_hbm, rel_hbm, out_hbm,
                 idxraw, idxpair, hrows, rrows, trows, outv, sem):
    wid = lax.axis_index("s") * 2 + lax.axis_index("c")

    pltpu.sync_copy(idx_hbm.at[wid], idxraw)

    def pair_ent(m, carry):
        row = m // 8 + (m // (8 * NCHUNK)) * NCHUNK
        sl = pl.ds((m % 8) * L, L)
        v = idxraw[row, sl]
        idxpair[row, sl] = ((v >> (TBL + 1)) << TBL) | (v & (TB - 1))
        return carry

    lax.fori_loop(0, 2 * NCHUNK * 8, pair_ent, 0)

    def pair_rel(m, carry):
        row = NCHUNK + m // 8
        sl = pl.ds((m % 8) * L, L)
        idxpair[row, sl] = idxraw[row, sl] >> 1
        return carry

    lax.fori_loop(0, NCHUNK * 8, pair_rel, 0)

    lane = lax.iota(jnp.int32, L)

    for rnd in range(NROUND):
        copies = []
        for c in range(RCH):
            cc = rnd * RCH + c
            dst = pl.ds(c * CH, CH)
            copies.append(pltpu.async_copy(
                ent_hbm.at[idxpair.at[cc]], hrows.at[dst], sem))
            copies.append(pltpu.async_copy(
                rel_hbm.at[idxpair.at[NCHUNK + cc]], rrows.at[dst], sem))
            copies.append(pltpu.async_copy(
                ent_hbm.at[idxpair.at[2 * NCHUNK + cc]], trows.at[dst], sem))
        for cp in copies:
            cp.wait()

        def group_body(g, carry):
            cc = rnd * RCH + g // 8
            psl = pl.ds((g % 8) * L, L)
            hqv = (idxraw[cc, psl] >> TBL) & 1
            rqv = idxraw[NCHUNK + cc, psl] & 1
            tqv = (idxraw[2 * NCHUNK + cc, psl] >> TBL) & 1
            svec = jnp.zeros((L,), jnp.float32)
            for k in range(L):
                i = g * L + k
                offh = hqv[k] * D
                offr = rqv[k] * D
                offt = tqv[k] * D
                acc = jnp.zeros((L,), jnp.float32)
                for j in range(D // L):
                    h = hrows[i, pl.ds(offh + j * L, L)]
                    r = rrows[i, pl.ds(offr + j * L, L)]
                    t = trows[i, pl.ds(offt + j * L, L)]
                    d = (h + r) - t
                    acc = acc + d * d
                svec = jnp.where(lane == k, jnp.sum(acc), svec)
            x = svec + 1e-12
            xi = plsc.bitcast(x, jnp.int32)
            yi = jnp.full((L,), 0x5F3759DF, jnp.int32) - (xi >> 1)
            y = plsc.bitcast(yi, jnp.float32)
            for _ in range(3):
                y = y * (1.5 - 0.5 * x * y * y)
            outv[pl.ds(rnd * RROWS + g * L, L)] = -(x * y)
            return carry

        lax.fori_loop(0, RROWS // L, group_body, 0)

    pltpu.sync_copy(outv, out_hbm.at[pl.ds(wid * BPW, BPW)])


@jax.jit
def _transe_sc(idx_all, ent2, rel2):
    mesh = plsc.VectorSubcoreMesh(core_axis_name="c", subcore_axis_name="s")
    return pl.kernel(
        _transe_body,
        mesh=mesh,
        compiler_params=pltpu.CompilerParams(
            needs_layout_passes=False, use_tc_tiling_on_sc=True),
        out_type=jax.ShapeDtypeStruct((B,), jnp.float32),
        scratch_types=[
            pltpu.VMEM((NT, CH), jnp.int32),
            pltpu.VMEM((NT, CH), jnp.int32),
            pltpu.VMEM((RROWS, W), jnp.float32),
            pltpu.VMEM((RROWS, W), jnp.float32),
            pltpu.VMEM((RROWS, W), jnp.float32),
            pltpu.VMEM((BPW,), jnp.float32),
            pltpu.SemaphoreType.DMA,
        ],
    )(idx_all, ent2, rel2)


def kernel(heads, relations, tails, entity_embed, relation_embed):
    ent2 = _fold_tc(entity_embed.T)
    rel2 = relation_embed.reshape(NR // 2, W)
    h32 = heads.astype(jnp.int32).reshape(NW, NCHUNK, CH)
    r32 = relations.astype(jnp.int32).reshape(NW, NCHUNK, CH)
    t32 = tails.astype(jnp.int32).reshape(NW, NCHUNK, CH)
    idx_all = jnp.concatenate([h32, r32, t32], axis=1)
    return _transe_sc(idx_all, ent2, rel2)


# speedup vs baseline: 3.2590x; 1.0215x over previous
import functools

import jax
import jax.numpy as jnp
from jax import lax
from jax.experimental import pallas as pl
from jax.experimental.pallas import tpu as pltpu
from jax.experimental.pallas import tpu_sc as plsc

B = 16384
D = 64
W = 2 * D
NE = 1000000
NR = 1000
NW = 32
BPW = B // NW
CH = 128
NCHUNK = BPW // CH
NROUND = 2
RCH = NCHUNK // NROUND
RROWS = RCH * CH
NT = 3 * NCHUNK
L = 16

TB = 16384
TBL = TB.bit_length() - 1
TGRID = (NE + 2 * TB - 1) // (2 * TB)
NP = TGRID * TB


def _fold_body(lo_ref, hi_ref, out_ref):
    z = jnp.concatenate([lo_ref[...], hi_ref[...]], axis=0)
    out_ref[...] = z.T


@jax.jit
def _fold_tc(entt):
    return pl.pallas_call(
        _fold_body,
        grid=(TGRID,),
        in_specs=[
            pl.BlockSpec((D, TB), lambda i: (0, 2 * i)),
            pl.BlockSpec(
                (D, TB),
                lambda i: (0, jnp.minimum(2 * i + 1, NE // TB))),
        ],
        out_specs=pl.BlockSpec((TB, W), lambda i: (i, 0)),
        out_shape=jax.ShapeDtypeStruct((NP, W), jnp.float32),
    )(entt, entt)


def _transe_body(idx_hbm, ent_hbm, rel_hbm, out_hbm,
                 idxraw, idxpair, hrows, rrows, trows, outv, sem):
    wid = lax.axis_index("s") * 2 + lax.axis_index("c")

    pltpu.sync_copy(idx_hbm.at[wid], idxraw)

    def pair_ent(m, carry):
        row = m // 8 + (m // (8 * NCHUNK)) * NCHUNK
        sl = pl.ds((m % 8) * L, L)
        v = idxraw[row, sl]
        idxpair[row, sl] = ((v >> (TBL + 1)) << TBL) | (v & (TB - 1))
        return carry

    lax.fori_loop(0, 2 * NCHUNK * 8, pair_ent, 0)

    def pair_rel(m, carry):
        row = NCHUNK + m // 8
        sl = pl.ds((m % 8) * L, L)
        idxpair[row, sl] = idxraw[row, sl] >> 1
        return carry

    lax.fori_loop(0, NCHUNK * 8, pair_rel, 0)

    lane = lax.iota(jnp.int32, L)

    def fire(c, buf):
        dst = pl.ds(buf * CH, CH)
        return [
            pltpu.async_copy(ent_hbm.at[idxpair.at[c]], hrows.at[dst], sem),
            pltpu.async_copy(
                rel_hbm.at[idxpair.at[NCHUNK + c]], rrows.at[dst], sem),
            pltpu.async_copy(
                ent_hbm.at[idxpair.at[2 * NCHUNK + c]], trows.at[dst], sem),
        ]

    def compute(c, buf):
        def group_body(g, carry):
            psl = pl.ds(g * L, L)
            hqv = (idxraw[c, psl] >> TBL) & 1
            rqv = idxraw[NCHUNK + c, psl] & 1
            tqv = (idxraw[2 * NCHUNK + c, psl] >> TBL) & 1
            svec = jnp.zeros((L,), jnp.float32)
            for k in range(L):
                i = buf * CH + g * L + k
                offh = hqv[k] * D
                offr = rqv[k] * D
                offt = tqv[k] * D
                acc = jnp.zeros((L,), jnp.float32)
                for j in range(D // L):
                    h = hrows[i, pl.ds(offh + j * L, L)]
                    r = rrows[i, pl.ds(offr + j * L, L)]
                    t = trows[i, pl.ds(offt + j * L, L)]
                    d = (h + r) - t
                    acc = acc + d * d
                svec = jnp.where(lane == k, jnp.sum(acc), svec)
            x = svec + 1e-12
            xi = plsc.bitcast(x, jnp.int32)
            yi = jnp.full((L,), 0x5F3759DF, jnp.int32) - (xi >> 1)
            y = plsc.bitcast(yi, jnp.float32)
            for _ in range(3):
                y = y * (1.5 - 0.5 * x * y * y)
            outv[pl.ds(c * CH + g * L, L)] = -(x * y)
            return carry

        lax.fori_loop(0, CH // L, group_body, 0)

    inflight = fire(0, 0)
    for c in range(NCHUNK):
        nxt = []
        if c + 1 < NCHUNK:
            nxt = fire(c + 1, (c + 1) % 2)
        for cp in inflight:
            cp.wait()
        compute(c, c % 2)
        inflight = nxt

    pltpu.sync_copy(outv, out_hbm.at[pl.ds(wid * BPW, BPW)])


@jax.jit
def _transe_sc(idx_all, ent2, rel2):
    mesh = plsc.VectorSubcoreMesh(core_axis_name="c", subcore_axis_name="s")
    return pl.kernel(
        _transe_body,
        mesh=mesh,
        compiler_params=pltpu.CompilerParams(
            needs_layout_passes=False, use_tc_tiling_on_sc=True),
        out_type=jax.ShapeDtypeStruct((B,), jnp.float32),
        scratch_types=[
            pltpu.VMEM((NT, CH), jnp.int32),
            pltpu.VMEM((NT, CH), jnp.int32),
            pltpu.VMEM((2 * CH, W), jnp.float32),
            pltpu.VMEM((2 * CH, W), jnp.float32),
            pltpu.VMEM((2 * CH, W), jnp.float32),
            pltpu.VMEM((BPW,), jnp.float32),
            pltpu.SemaphoreType.DMA,
        ],
    )(idx_all, ent2, rel2)


def kernel(heads, relations, tails, entity_embed, relation_embed):
    ent2 = _fold_tc(entity_embed.T)
    rel2 = relation_embed.reshape(NR // 2, W)
    h32 = heads.astype(jnp.int32).reshape(NW, NCHUNK, CH)
    r32 = relations.astype(jnp.int32).reshape(NW, NCHUNK, CH)
    t32 = tails.astype(jnp.int32).reshape(NW, NCHUNK, CH)
    idx_all = jnp.concatenate([h32, r32, t32], axis=1)
    return _transe_sc(idx_all, ent2, rel2)
